# baseline (speedup 1.0000x reference)
import functools

import jax
import jax.numpy as jnp
from jax import lax
from jax.experimental import pallas as pl
from jax.experimental.pallas import tpu as pltpu
from jax.experimental.pallas import tpu_sc as plsc


def _sc_gather(flat_idx, table):
    R = flat_idx.shape[0]
    D = table.shape[1]
    info = plsc.get_sparse_core_info()
    nw = info.num_cores * info.num_subcores
    per_w = R // nw
    ch = 80
    mesh = plsc.VectorSubcoreMesh(core_axis_name="c", subcore_axis_name="s")

    @functools.partial(
        pl.kernel,
        mesh=mesh,
        out_type=jax.ShapeDtypeStruct((R, D), jnp.float32),
        scratch_types=[
            pltpu.VMEM((per_w,), jnp.int32),
            pltpu.VMEM((ch, D), jnp.float32),
            pltpu.SemaphoreType.DMA,
        ],
    )
    def gk(idx_hbm, tab_hbm, out_hbm, idx_v, rows_v, sem):
        wid = lax.axis_index("s") * info.num_cores + lax.axis_index("c")
        base = wid * per_w
        pltpu.sync_copy(idx_hbm.at[pl.ds(base, per_w)], idx_v)
        for j in range(per_w // ch):
            pltpu.async_copy(tab_hbm.at[idx_v.at[pl.ds(j * ch, ch)]],
                             rows_v, sem).wait()
            pltpu.sync_copy(rows_v, out_hbm.at[pl.ds(base + j * ch, ch)])

    return gk(flat_idx, table)


def _tc_body(bb, L, Wc, CV,
             cidx_ref, wemb_ref, ctab_ref, cw_ref, cb_ref, pw_ref,
             g0w_ref, g0b_ref, t0w_ref, t0b_ref,
             g1w_ref, g1b_ref, t1w_ref, t1b_ref, out_ref):
    n = L * Wc
    ctab = ctab_ref[...]
    taps = [
        lax.dot_general(ctab, cw_ref[k], (((1,), (1,)), ((), ())),
                        preferred_element_type=jnp.float32)
        for k in range(3)
    ]
    P = jnp.concatenate(taps, axis=1).astype(jnp.bfloat16)

    idx = cidx_ref[...]
    mask = (idx[:, :, None]
            == lax.broadcasted_iota(jnp.int32, (bb, n, CV), 2))
    oh = jnp.where(mask, 1.0, 0.0).astype(jnp.bfloat16).reshape(bb * n, CV)
    T = jnp.dot(oh, P, preferred_element_type=jnp.float32)
    T = T.reshape(bb, n, 192)
    t0 = T[:, :, 0:64]
    t1 = T[:, :, 64:128]
    t2 = T[:, :, 128:192]
    z = jnp.zeros((bb, 1, 64), jnp.float32)
    conv = (jnp.concatenate([z, t0[:, :-1, :]], axis=1)
            + t1
            + jnp.concatenate([t2[:, 1:, :], z], axis=1)
            + cb_ref[...][None])
    conv = jnp.maximum(conv, 0.0)
    cf = jnp.max(conv.reshape(bb, L, Wc, 64), axis=2)
    cf = cf.reshape(bb * L, 64)

    wp = jnp.dot(wemb_ref[...], pw_ref[...],
                 preferred_element_type=jnp.float32)
    emb = jnp.concatenate([wp, cf], axis=1)

    for gw_ref, gb_ref, tw_ref, tb_ref in (
            (g0w_ref, g0b_ref, t0w_ref, t0b_ref),
            (g1w_ref, g1b_ref, t1w_ref, t1b_ref)):
        gpre = jnp.dot(emb, gw_ref[...],
                       preferred_element_type=jnp.float32) + gb_ref[...]
        tpre = jnp.dot(emb, tw_ref[...],
                       preferred_element_type=jnp.float32) + tb_ref[...]
        g = 1.0 / (1.0 + jnp.exp(-gpre))
        t = jnp.maximum(tpre, 0.0)
        emb = g * t + (1.0 - g) * emb

    out_ref[...] = emb


def kernel(word_idxs, char_idxs, word_table, char_table, conv_w, conv_b,
           proj_w, g0_w, g0_b, t0_w, t0_b, g1_w, g1_b, t1_w, t1_b):
    B, L = word_idxs.shape
    Wc = char_idxs.shape[2]
    CV = char_table.shape[0]
    HID = g0_w.shape[0]

    word_emb = _sc_gather(word_idxs.reshape(-1), word_table)

    cidx2 = char_idxs.reshape(B, L * Wc)
    cw3 = jnp.transpose(conv_w, (2, 0, 1))
    cb2 = conv_b.reshape(1, -1)
    g0b2, t0b2 = g0_b.reshape(1, -1), t0_b.reshape(1, -1)
    g1b2, t1b2 = g1_b.reshape(1, -1), t1_b.reshape(1, -1)

    bb = 8
    n = L * Wc
    grid = (B // bb,)
    body = functools.partial(_tc_body, bb, L, Wc, CV)
    out = pl.pallas_call(
        body,
        grid=grid,
        in_specs=[
            pl.BlockSpec((bb, n), lambda i: (i, 0)),
            pl.BlockSpec((bb * L, HID), lambda i: (i, 0)),
            pl.BlockSpec(char_table.shape, lambda i: (0, 0)),
            pl.BlockSpec(cw3.shape, lambda i: (0, 0, 0)),
            pl.BlockSpec(cb2.shape, lambda i: (0, 0)),
            pl.BlockSpec(proj_w.shape, lambda i: (0, 0)),
            pl.BlockSpec(g0_w.shape, lambda i: (0, 0)),
            pl.BlockSpec(g0b2.shape, lambda i: (0, 0)),
            pl.BlockSpec(t0_w.shape, lambda i: (0, 0)),
            pl.BlockSpec(t0b2.shape, lambda i: (0, 0)),
            pl.BlockSpec(g1_w.shape, lambda i: (0, 0)),
            pl.BlockSpec(g1b2.shape, lambda i: (0, 0)),
            pl.BlockSpec(t1_w.shape, lambda i: (0, 0)),
            pl.BlockSpec(t1b2.shape, lambda i: (0, 0)),
        ],
        out_specs=pl.BlockSpec((bb * L, HID), lambda i: (i, 0)),
        out_shape=jax.ShapeDtypeStruct((B * L, HID), jnp.float32),
    )(cidx2, word_emb, char_table, cw3, cb2, proj_w,
      g0_w, g0b2, t0_w, t0b2, g1_w, g1b2, t1_w, t1b2)

    return out.reshape(B, L, HID)


# speedup vs baseline: 1.0610x; 1.0610x over previous
import functools

import jax
import jax.numpy as jnp
from jax import lax
from jax.experimental import pallas as pl
from jax.experimental.pallas import tpu as pltpu
from jax.experimental.pallas import tpu_sc as plsc


def _sc_gather(flat_idx, table):
    R = flat_idx.shape[0]
    D = table.shape[1]
    info = plsc.get_sparse_core_info()
    nw = info.num_cores * info.num_subcores
    per_w = R // nw
    ch = 80
    mesh = plsc.VectorSubcoreMesh(core_axis_name="c", subcore_axis_name="s")

    @functools.partial(
        pl.kernel,
        mesh=mesh,
        out_type=jax.ShapeDtypeStruct((R, D), jnp.float32),
        scratch_types=[
            pltpu.VMEM((per_w,), jnp.int32),
            pltpu.VMEM((ch, D), jnp.float32),
            pltpu.SemaphoreType.DMA,
        ],
    )
    def gk(idx_hbm, tab_hbm, out_hbm, idx_v, rows_v, sem):
        wid = lax.axis_index("s") * info.num_cores + lax.axis_index("c")
        base = wid * per_w
        pltpu.sync_copy(idx_hbm.at[pl.ds(base, per_w)], idx_v)
        for j in range(per_w // ch):
            pltpu.async_copy(tab_hbm.at[idx_v.at[pl.ds(j * ch, ch)]],
                             rows_v, sem).wait()
            pltpu.sync_copy(rows_v, out_hbm.at[pl.ds(base + j * ch, ch)])

    return gk(flat_idx, table)


def _char_body(bb, L, Wc, CV, cidx_ref, ctab_ref, cw_ref, cb_ref, out_ref):
    n = L * Wc
    ctab = ctab_ref[...]
    taps = [
        lax.dot_general(ctab, cw_ref[k], (((1,), (1,)), ((), ())),
                        preferred_element_type=jnp.float32)
        for k in range(3)
    ]
    P = jnp.concatenate(taps, axis=1).astype(jnp.bfloat16)

    idxT = jnp.transpose(cidx_ref[...]).astype(jnp.int16)
    iota = lax.broadcasted_iota(jnp.int16, (n, CV), 1)
    cb = cb_ref[...]
    z = jnp.zeros((1, 64), jnp.float32)
    for b in range(bb):
        col = lax.slice(idxT, (0, b), (n, b + 1))
        oh = jnp.where(col == iota, jnp.bfloat16(1.0),
                       jnp.bfloat16(0.0))
        T = jnp.dot(oh, P, preferred_element_type=jnp.float32)
        conv = (jnp.concatenate([z, T[:-1, 0:64]], axis=0)
                + T[:, 64:128]
                + jnp.concatenate([T[1:, 128:192], z], axis=0)
                + cb)
        conv = jnp.maximum(conv, 0.0)
        out_ref[b * L:(b + 1) * L, :] = jnp.max(
            conv.reshape(L, Wc, 64), axis=1)


def _fuse_body(cf_ref, wemb_ref, pw_ref,
               gt0w_ref, gt0b_ref, gt1w_ref, gt1b_ref, out_ref):
    wp = jnp.dot(wemb_ref[...], pw_ref[...],
                 preferred_element_type=jnp.float32)
    emb = jnp.concatenate([wp, cf_ref[...]], axis=1)
    h = emb.shape[1]
    for gtw_ref, gtb_ref in ((gt0w_ref, gt0b_ref), (gt1w_ref, gt1b_ref)):
        pre = jnp.dot(emb, gtw_ref[...],
                      preferred_element_type=jnp.float32) + gtb_ref[...]
        g = 1.0 / (1.0 + jnp.exp(-pre[:, :h]))
        t = jnp.maximum(pre[:, h:], 0.0)
        emb = g * t + (1.0 - g) * emb
    out_ref[...] = emb


def kernel(word_idxs, char_idxs, word_table, char_table, conv_w, conv_b,
           proj_w, g0_w, g0_b, t0_w, t0_b, g1_w, g1_b, t1_w, t1_b):
    B, L = word_idxs.shape
    Wc = char_idxs.shape[2]
    CV = char_table.shape[0]
    HID = g0_w.shape[0]

    word_emb = _sc_gather(word_idxs.reshape(-1), word_table)

    cidx2 = char_idxs.reshape(B, L * Wc)
    cw3 = jnp.transpose(conv_w, (2, 0, 1))
    cb2 = conv_b.reshape(1, -1)
    gt0_w = jnp.concatenate([g0_w, t0_w], axis=1)
    gt1_w = jnp.concatenate([g1_w, t1_w], axis=1)
    gt0_b = jnp.concatenate([g0_b, t0_b]).reshape(1, -1)
    gt1_b = jnp.concatenate([g1_b, t1_b]).reshape(1, -1)

    bb = 8
    n = L * Wc
    cf = pl.pallas_call(
        functools.partial(_char_body, bb, L, Wc, CV),
        grid=(B // bb,),
        in_specs=[
            pl.BlockSpec((bb, n), lambda i: (i, 0)),
            pl.BlockSpec(char_table.shape, lambda i: (0, 0)),
            pl.BlockSpec(cw3.shape, lambda i: (0, 0, 0)),
            pl.BlockSpec(cb2.shape, lambda i: (0, 0)),
        ],
        out_specs=pl.BlockSpec((bb * L, HID // 2), lambda i: (i, 0)),
        out_shape=jax.ShapeDtypeStruct((B * L, HID // 2), jnp.float32),
    )(cidx2, char_table, cw3, cb2)

    bf = 64
    out = pl.pallas_call(
        _fuse_body,
        grid=(B // bf,),
        in_specs=[
            pl.BlockSpec((bf * L, HID // 2), lambda i: (i, 0)),
            pl.BlockSpec((bf * L, HID), lambda i: (i, 0)),
            pl.BlockSpec(proj_w.shape, lambda i: (0, 0)),
            pl.BlockSpec(gt0_w.shape, lambda i: (0, 0)),
            pl.BlockSpec(gt0_b.shape, lambda i: (0, 0)),
            pl.BlockSpec(gt1_w.shape, lambda i: (0, 0)),
            pl.BlockSpec(gt1_b.shape, lambda i: (0, 0)),
        ],
        out_specs=pl.BlockSpec((bf * L, HID), lambda i: (i, 0)),
        out_shape=jax.ShapeDtypeStruct((B * L, HID), jnp.float32),
    )(cf, word_emb, proj_w, gt0_w, gt0_b, gt1_w, gt1_b)

    return out.reshape(B, L, HID)
